# baseline (speedup 1.0000x reference)
import functools

import jax
import jax.numpy as jnp
from jax.experimental import pallas as pl

HIDDEN = 128
HEADS = 4
DH = HIDDEN // HEADS
N_LAYERS = 3
NODE_TYPES = ["transaction", "card"]
EDGE_TYPES = [("transaction", "to", "card"), ("card", "rev_to", "transaction")]
N_NODES = {"transaction": 100000, "card": 10000}


def _layer_norm(x, s, b, eps=1e-5):
    mu = jnp.mean(x, axis=-1, keepdims=True)
    var = jnp.mean((x - mu) ** 2, axis=-1, keepdims=True)
    return (x - mu) / jnp.sqrt(var + eps) * s + b


def _elu(x):
    return jnp.where(x > 0, x, jnp.expm1(x))


def _elu_k(x):
    return jnp.where(x > 0, x, jnp.exp(jnp.minimum(x, 0.0)) - 1.0)


def _leaky_relu(x, a=0.2):
    return jnp.where(x > 0, x, a * x)


def _gat_conv(h_src, h_dst, src_idx, dst_idx, Wsrc, Wdst, asrc, adst, b, n_dst):
    hs = (h_src @ Wsrc).reshape(-1, HEADS, DH)
    hd = (h_dst @ Wdst).reshape(-1, HEADS, DH)
    al_s = jnp.sum(hs * asrc, axis=-1)
    al_d = jnp.sum(hd * adst, axis=-1)
    e = _leaky_relu(al_s[src_idx] + al_d[dst_idx])
    emax = jax.ops.segment_max(e, dst_idx, num_segments=n_dst)
    emax = jax.lax.stop_gradient(jnp.where(jnp.isfinite(emax), emax, 0.0))
    expe = jnp.exp(e - emax[dst_idx])
    denom = jax.ops.segment_sum(expe, dst_idx, num_segments=n_dst)
    attn = expe / (denom[dst_idx] + 1e-16)
    msg = hs[src_idx] * attn[:, :, None]
    out = jax.ops.segment_sum(msg, dst_idx, num_segments=n_dst)
    return out.reshape(n_dst, HIDDEN) + b


def _head_body(h_ref, embW_ref, embb_ref, lns_ref, lnb_ref, W1_ref, b1_ref,
               W2_ref, b2_ref, out_ref):
    h = h_ref[...]
    z = jnp.dot(h, embW_ref[...], preferred_element_type=jnp.float32) + embb_ref[...]
    z = _elu_k(_layer_norm(z, lns_ref[...], lnb_ref[...]))
    hc = jnp.maximum(jnp.dot(z, W1_ref[...], preferred_element_type=jnp.float32)
                     + b1_ref[...], 0.0)
    out_ref[...] = hc @ W2_ref[...] + b2_ref[...]


def _head(h_t, p):
    blk = 2000
    n = h_t.shape[0]
    grid = n // blk
    out = pl.pallas_call(
        _head_body,
        grid=(grid,),
        in_specs=[
            pl.BlockSpec((blk, HIDDEN), lambda i: (i, 0)),
            pl.BlockSpec((HIDDEN, 64), lambda i: (0, 0)),
            pl.BlockSpec((64,), lambda i: (0,)),
            pl.BlockSpec((64,), lambda i: (0,)),
            pl.BlockSpec((64,), lambda i: (0,)),
            pl.BlockSpec((64, 32), lambda i: (0, 0)),
            pl.BlockSpec((32,), lambda i: (0,)),
            pl.BlockSpec((32, 1), lambda i: (0, 0)),
            pl.BlockSpec((1,), lambda i: (0,)),
        ],
        out_specs=pl.BlockSpec((blk, 1), lambda i: (i, 0)),
        out_shape=jax.ShapeDtypeStruct((n, 1), jnp.float32),
    )(h_t, p["emb_W"], p["emb_b"], p["emb_ln_s"], p["emb_ln_b"],
      p["cls_W1"], p["cls_b1"], p["cls_W2"], p["cls_b2"])
    return out[:, 0]


def kernel(x_transaction, x_card, params, edge_index_tc, edge_index_ct):
    x = {"transaction": x_transaction, "card": x_card}
    edges = {EDGE_TYPES[0]: edge_index_tc, EDGE_TYPES[1]: edge_index_ct}
    h = {}
    for nt in NODE_TYPES:
        z = x[nt] @ params["proj_%s_W" % nt] + params["proj_%s_b" % nt]
        h[nt] = _elu(_layer_norm(z, params["proj_%s_ln_s" % nt], params["proj_%s_ln_b" % nt]))
    for l in range(N_LAYERS):
        agg = {nt: jnp.zeros_like(h[nt]) for nt in NODE_TYPES}
        for (s, r, d) in EDGE_TYPES:
            ei = edges[(s, r, d)]
            p = "l%d_%s_%s_%s" % (l, s, r, d)
            agg[d] = agg[d] + _gat_conv(h[s], h[d], ei[0], ei[1],
                                        params[p + "_Wsrc"], params[p + "_Wdst"],
                                        params[p + "_asrc"], params[p + "_adst"],
                                        params[p + "_b"], N_NODES[d])
        h = {nt: _elu(_layer_norm(agg[nt] + h[nt],
                                  params["l%d_%s_ln_s" % (l, nt)],
                                  params["l%d_%s_ln_b" % (l, nt)])) for nt in NODE_TYPES}
    return _head(h["transaction"], params)


# speedup vs baseline: 37.3556x; 37.3556x over previous
import functools

import jax
import jax.numpy as jnp
import numpy as np
from jax import lax
from jax.experimental import pallas as pl
from jax.experimental.pallas import tpu as pltpu
from jax.experimental.pallas import tpu_sc as plsc

HIDDEN = 128
HEADS = 4
DH = HIDDEN // HEADS
N_LAYERS = 3
N_T = 100000
N_C = 10000
E = 300000
BE = 16
ZB = 64
NSC = 2
NTILE = 16
E_PAD = E + BE

CFG_T = (N_T, 8192, 13)
CFG_C = (N_C, 5120, 2)


def _layer_norm(x, s, b, eps=1e-5):
    mu = jnp.mean(x, axis=-1, keepdims=True)
    var = jnp.mean((x - mu) ** 2, axis=-1, keepdims=True)
    return (x - mu) / jnp.sqrt(var + eps) * s + b


def _elu_k(x):
    return jnp.where(x > 0, x, jnp.exp(jnp.minimum(x, 0.0)) - 1.0)


def _proj_body(x_ref, W_ref, b_ref, s_ref, bb_ref, out_ref):
    z = jnp.dot(x_ref[...], W_ref[...], preferred_element_type=jnp.float32)
    out_ref[...] = _elu_k(_layer_norm(z + b_ref[...], s_ref[...], bb_ref[...]))


def _proj(x, W, b, s, bb):
    n = x.shape[0]
    blk = 2000
    return pl.pallas_call(
        _proj_body,
        grid=(n // blk,),
        in_specs=[
            pl.BlockSpec((blk, HIDDEN), lambda i: (i, 0)),
            pl.BlockSpec((HIDDEN, HIDDEN), lambda i: (0, 0)),
            pl.BlockSpec((HIDDEN,), lambda i: (0,)),
            pl.BlockSpec((HIDDEN,), lambda i: (0,)),
            pl.BlockSpec((HIDDEN,), lambda i: (0,)),
        ],
        out_specs=pl.BlockSpec((blk, HIDDEN), lambda i: (i, 0)),
        out_shape=jax.ShapeDtypeStruct((n, HIDDEN), jnp.float32),
    )(x, W, b, s, bb)


def _prep_body(h_ref, W_ref, A1_ref, A2_ref, hs_ref, als_ref, ald_ref):
    h = h_ref[...]
    hs_ref[...] = jnp.dot(h, W_ref[...], preferred_element_type=jnp.float32)
    als_ref[...] = jnp.dot(h, A1_ref[...], preferred_element_type=jnp.float32)
    ald_ref[...] = jnp.dot(h, A2_ref[...], preferred_element_type=jnp.float32)


def _prep(h, W, A1, A2):
    n = h.shape[0]
    blk = 2000
    return pl.pallas_call(
        _prep_body,
        grid=(n // blk,),
        in_specs=[
            pl.BlockSpec((blk, HIDDEN), lambda i: (i, 0)),
            pl.BlockSpec((HIDDEN, HIDDEN), lambda i: (0, 0)),
            pl.BlockSpec((HIDDEN, 16), lambda i: (0, 0)),
            pl.BlockSpec((HIDDEN, 16), lambda i: (0, 0)),
        ],
        out_specs=[
            pl.BlockSpec((blk, HIDDEN), lambda i: (i, 0)),
            pl.BlockSpec((blk, 16), lambda i: (i, 0)),
            pl.BlockSpec((blk, 16), lambda i: (i, 0)),
        ],
        out_shape=[
            jax.ShapeDtypeStruct((n, HIDDEN), jnp.float32),
            jax.ShapeDtypeStruct((n, 16), jnp.float32),
            jax.ShapeDtypeStruct((n, 16), jnp.float32),
        ],
    )(h, W, A1, A2)


def _post_body(raw_ref, den_ref, hp_ref, S_ref, b_ref, s_ref, bb_ref, out_ref):
    d = jnp.dot(den_ref[...], S_ref[...], preferred_element_type=jnp.float32)
    x = raw_ref[...] / (d + 1e-16) + b_ref[...] + hp_ref[...]
    out_ref[...] = _elu_k(_layer_norm(x, s_ref[...], bb_ref[...]))


def _post(raw, den, hp, S, b, s, bb):
    n = raw.shape[0]
    blk = 2000
    return pl.pallas_call(
        _post_body,
        grid=(n // blk,),
        in_specs=[
            pl.BlockSpec((blk, HIDDEN), lambda i: (i, 0)),
            pl.BlockSpec((blk, 16), lambda i: (i, 0)),
            pl.BlockSpec((blk, HIDDEN), lambda i: (i, 0)),
            pl.BlockSpec((16, HIDDEN), lambda i: (0, 0)),
            pl.BlockSpec((HIDDEN,), lambda i: (0,)),
            pl.BlockSpec((HIDDEN,), lambda i: (0,)),
            pl.BlockSpec((HIDDEN,), lambda i: (0,)),
        ],
        out_specs=pl.BlockSpec((blk, HIDDEN), lambda i: (i, 0)),
        out_shape=jax.ShapeDtypeStruct((n, HIDDEN), jnp.float32),
    )(raw, den, hp, S, b, s, bb)


def _head_body(h_ref, embW_ref, embb_ref, lns_ref, lnb_ref, W1_ref, b1_ref,
               W2_ref, b2_ref, out_ref):
    h = h_ref[...]
    z = jnp.dot(h, embW_ref[...], preferred_element_type=jnp.float32) + embb_ref[...]
    z = _elu_k(_layer_norm(z, lns_ref[...], lnb_ref[...]))
    hc = jnp.maximum(jnp.dot(z, W1_ref[...], preferred_element_type=jnp.float32)
                     + b1_ref[...], 0.0)
    out_ref[...] = hc @ W2_ref[...] + b2_ref[...]


def _head(h_t, p):
    blk = 2000
    n = h_t.shape[0]
    out = pl.pallas_call(
        _head_body,
        grid=(n // blk,),
        in_specs=[
            pl.BlockSpec((blk, HIDDEN), lambda i: (i, 0)),
            pl.BlockSpec((HIDDEN, 64), lambda i: (0, 0)),
            pl.BlockSpec((64,), lambda i: (0,)),
            pl.BlockSpec((64,), lambda i: (0,)),
            pl.BlockSpec((64,), lambda i: (0,)),
            pl.BlockSpec((64, 32), lambda i: (0, 0)),
            pl.BlockSpec((32,), lambda i: (0,)),
            pl.BlockSpec((32, 1), lambda i: (0, 0)),
            pl.BlockSpec((1,), lambda i: (0,)),
        ],
        out_specs=pl.BlockSpec((blk, 1), lambda i: (i, 0)),
        out_shape=jax.ShapeDtypeStruct((n, 1), jnp.float32),
    )(h_t, p["emb_W"], p["emb_b"], p["emb_ln_s"], p["emb_ln_b"],
      p["cls_W1"], p["cls_b1"], p["cls_W2"], p["cls_b2"])
    return out[:, 0]


@functools.lru_cache(maxsize=None)
def _make_conv(n_src, ch, nc):
    n_pad = ch * nc
    zr = ch // NTILE
    nz = zr // ZB
    dump = ch
    mesh = plsc.VectorSubcoreMesh(core_axis_name="c", subcore_axis_name="s")

    @functools.partial(
        pl.kernel,
        mesh=mesh,
        compiler_params=pltpu.CompilerParams(use_tc_tiling_on_sc=False),
        out_type=[
            jax.ShapeDtypeStruct((n_pad, HIDDEN), jnp.float32),
            jax.ShapeDtypeStruct((n_pad, 16), jnp.float32),
        ],
        scratch_types=[
            pltpu.VMEM((16,), jnp.int32),
            pltpu.SMEM((16,), jnp.int32),
            pltpu.VMEM((BE,), jnp.int32),
            pltpu.VMEM((BE,), jnp.int32),
            pltpu.VMEM((BE,), jnp.int32),
            pltpu.VMEM((BE, 16), jnp.float32),
            pltpu.VMEM((BE, 16), jnp.float32),
            pltpu.VMEM((BE, 16), jnp.float32),
            pltpu.VMEM((BE, HIDDEN), jnp.float32),
            pltpu.VMEM((ZB, HIDDEN), jnp.float32),
            pltpu.VMEM((ZB, HIDDEN), jnp.float32),
            pltpu.VMEM((ZB, 16), jnp.float32),
            pltpu.VMEM((ZB, 16), jnp.float32),
            pltpu.VMEM_SHARED((ch + 32, HIDDEN), jnp.float32),
            pltpu.VMEM_SHARED((ch + 32, 16), jnp.float32),
            pltpu.SemaphoreType.DMA,
            pltpu.SemaphoreType.DMA,
            pltpu.SemaphoreType.DMA,
        ],
    )
    def conv(src_hbm, dst_hbm, bounds_hbm, hs_hbm, als_hbm, ald_hbm,
             out_hbm, den_hbm,
             bounds_v, bounds_sm, srcv, dstv, dstl, als_v, ald_v, expe_v,
             rows_v, zrow_v, drow_v, zden_v, dden_v, out_sh, den_sh,
             s1, s2, s3):
        scid = lax.axis_index("c")
        tid = lax.axis_index("s")
        pltpu.sync_copy(bounds_hbm, bounds_v)
        bv = bounds_v[...]
        for i in range(nc + 1):
            bounds_sm[i] = bv[i]

        z16 = jnp.zeros((16,), jnp.float32)

        def zb_body(r, _):
            for k in range(HIDDEN // 16):
                zrow_v[r, pl.ds(k * 16, 16)] = z16
            zden_v[r, :] = z16
            return 0

        lax.fori_loop(0, ZB, zb_body, 0)

        def batch_body(lo, hi, base, i, _):
            k = tid + i * NTILE
            e0 = (lo // 8) * 8 + k * BE
            c1 = pltpu.async_copy(src_hbm.at[pl.ds(e0, BE)], srcv, s1)
            c2 = pltpu.async_copy(dst_hbm.at[pl.ds(e0, BE)], dstv, s2)
            c1.wait()
            c2.wait()
            g1 = pltpu.async_copy(hs_hbm.at[srcv], rows_v, s1)
            g2 = pltpu.async_copy(als_hbm.at[srcv], als_v, s2)
            g3 = pltpu.async_copy(ald_hbm.at[dstv], ald_v, s3)
            pos = e0 + lax.iota(jnp.int32, 16)
            valid = (pos >= lo) & (pos < hi)
            dstl[...] = jnp.where(valid, dstv[...] - base, dump)
            fvec = jnp.where(valid, 1.0, 0.0)
            g2.wait()
            g3.wait()
            g1.wait()
            for t in range(BE):
                ev = als_v[t, :] + ald_v[t, :]
                ev = jnp.where(ev > 0, ev, 0.2 * ev)
                ew = jnp.exp(ev) * fvec[t]
                expe_v[t, :] = ew
                for h in range(HEADS):
                    w = ew[h]
                    rows_v[t, pl.ds(h * 2 * 16, 16)] = (
                        rows_v[t, pl.ds(h * 2 * 16, 16)] * w)
                    rows_v[t, pl.ds((h * 2 + 1) * 16, 16)] = (
                        rows_v[t, pl.ds((h * 2 + 1) * 16, 16)] * w)
            pltpu.sync_copy(expe_v, den_sh.at[dstl], add=True)
            pltpu.sync_copy(rows_v, out_sh.at[dstl], add=True)
            return 0

        def chunk_body(ci, _):
            c = scid + ci * NSC
            lo = bounds_sm[c]
            hi = bounds_sm[c + 1]
            base = c * ch
            for z in range(nz):
                r0 = (tid * nz + z) * ZB
                pltpu.sync_copy(zrow_v, out_sh.at[pl.ds(r0, ZB)])
                pltpu.sync_copy(zden_v, den_sh.at[pl.ds(r0, ZB)])
            plsc.subcore_barrier()
            e8 = (lo // 8) * 8
            nb = (hi - e8 + BE - 1) // BE
            nbt = (nb - tid + NTILE - 1) // NTILE
            lax.fori_loop(0, nbt,
                          functools.partial(batch_body, lo, hi, base), 0)
            plsc.subcore_barrier()
            for z in range(nz):
                r0 = (tid * nz + z) * ZB
                pltpu.sync_copy(out_sh.at[pl.ds(r0, ZB)], drow_v)
                pltpu.sync_copy(drow_v, out_hbm.at[pl.ds(base + r0, ZB)])
                pltpu.sync_copy(den_sh.at[pl.ds(r0, ZB)], dden_v)
                pltpu.sync_copy(dden_v, den_hbm.at[pl.ds(base + r0, ZB)])
            plsc.subcore_barrier()
            return 0

        nc_mine = (nc - scid + NSC - 1) // NSC
        lax.fori_loop(0, nc_mine, chunk_body, 0)

    return conv


_S128 = np.zeros((16, HIDDEN), np.float32)
for _h in range(HEADS):
    _S128[_h, _h * DH:(_h + 1) * DH] = 1.0

_I416 = np.eye(4, 16, dtype=np.float32)


def _fold_a(W, a):
    Em = (a[:, :, None] * _I416[:, None, :]).reshape(HIDDEN, 16)
    return W @ Em


def _edge_prep(ei, n_dst, ch, nc):
    dst, src = lax.sort([ei[1], ei[0]], num_keys=1)
    src = jnp.concatenate([src, jnp.zeros((E_PAD - E,), jnp.int32)])
    dst_p = jnp.concatenate([dst, jnp.zeros((E_PAD - E,), jnp.int32)])
    edges = jnp.arange(nc + 1, dtype=jnp.int32) * ch
    bounds = jnp.searchsorted(dst, edges).astype(jnp.int32)
    bounds = jnp.concatenate(
        [bounds, jnp.zeros((16 - nc - 1,), jnp.int32)])
    return src, dst_p, bounds


def kernel(x_transaction, x_card, params, edge_index_tc, edge_index_ct):
    p = params
    h = {}
    for nt in ("transaction", "card"):
        x = x_transaction if nt == "transaction" else x_card
        h[nt] = _proj(x, p["proj_%s_W" % nt], p["proj_%s_b" % nt],
                      p["proj_%s_ln_s" % nt], p["proj_%s_ln_b" % nt])

    S = jnp.asarray(_S128)
    cfgs = {
        "transaction": CFG_T,
        "card": CFG_C,
    }
    eprep = {
        "card": _edge_prep(edge_index_tc, *CFG_C),
        "transaction": _edge_prep(edge_index_ct, *CFG_T),
    }
    rel = {"card": ("transaction", "to", "card"),
           "transaction": ("card", "rev_to", "transaction")}
    n_nodes = {"transaction": N_T, "card": N_C}

    for l in range(N_LAYERS):
        raw, den = {}, {}
        for d in ("card", "transaction"):
            s, r, _ = rel[d]
            pref = "l%d_%s_%s_%s" % (l, s, r, d)
            A1 = _fold_a(p[pref + "_Wsrc"], p[pref + "_asrc"])
            A2 = _fold_a(p[pref + "_Wdst"], p[pref + "_adst"])
            hs, als, _ = _prep(h[s], p[pref + "_Wsrc"], A1, A2)
            _, _, ald = _prep(h[d], p[pref + "_Wdst"], A2, A2)
            n_dst, ch, nc = cfgs[d]
            conv = _make_conv(n_nodes[s], ch, nc)
            src_e, dst_e, bounds = eprep[d]
            out_pad, den_pad = conv(src_e, dst_e, bounds, hs, als, ald)
            raw[d] = out_pad[:n_dst]
            den[d] = den_pad[:n_dst]
        h = {d: _post(raw[d], den[d], h[d], S,
                      p["l%d_%s_%s_%s_b" % ((l,) + rel[d])],
                      p["l%d_%s_ln_s" % (l, d)], p["l%d_%s_ln_b" % (l, d)])
             for d in ("card", "transaction")}

    return _head(h["transaction"], p)
